# baseline (speedup 1.0000x reference)
import functools

import jax
import jax.numpy as jnp
from jax import lax
from jax.experimental import pallas as pl
from jax.experimental.pallas import tpu as pltpu
from jax.experimental.pallas import tpu_sc as plsc

N = 10000
NPAD = 10240
HALF = NPAD // 2
SPR = HALF // 16
GROWS = HALF + 16
TRASH = HALF
FB = 128

B = 128
SB = 16
NSB = 10
EPT = B * SB * NSB
E_PAD = 16 * EPT
EROWS = E_PAD // B

_mesh = lambda: plsc.VectorSubcoreMesh(core_axis_name="c", subcore_axis_name="s")


def _make_prop():
    @functools.partial(
        pl.kernel,
        out_type=jax.ShapeDtypeStruct((NPAD, 128), jnp.float32),
        mesh=_mesh(),
        scratch_types=[
            pltpu.VMEM_SHARED((GROWS, 128), jnp.float32),
            pltpu.VMEM_SHARED((GROWS, 128), jnp.float32),
            pltpu.VMEM((SB, B), jnp.int32),
            pltpu.VMEM((SB, B), jnp.int32),
            pltpu.VMEM((SB, B), jnp.int32),
            pltpu.VMEM((SB, B), jnp.int32),
            pltpu.VMEM((B, 128), jnp.float32),
            pltpu.VMEM((B, 128), jnp.float32),
            pltpu.SemaphoreType.DMA,
            pltpu.SemaphoreType.DMA,
            pltpu.SemaphoreType.DMA,
            pltpu.SemaphoreType.DMA,
            pltpu.SemaphoreType.DMA,
        ],
    )
    def k(g_hbm, src_hbm, dst_hbm, scale_hbm, out_hbm,
          gsp, acc, ssrc0, ssrc1, sdst0, sdst1,
          rows0, rows1, sg0, sg1, ss0, ss1, si):
        c = lax.axis_index("c")
        s = lax.axis_index("s")
        rb = (rows0, rows1)
        sg = (sg0, sg1)
        ss = (ss0, ss1)
        ssrcs = (ssrc0, ssrc1)
        sdsts = (sdst0, sdst1)
        rbase = s * SPR
        dstbase = c * HALF

        def zrow(i, _):
            for kk in range(8):
                rows1[i, pl.ds(kk * 16, 16)] = jnp.zeros((16,), jnp.float32)
            return 0

        lax.fori_loop(0, FB, zrow, 0)
        pltpu.sync_copy(rows1, acc.at[pl.ds(rbase, FB)])
        pltpu.sync_copy(rows1, acc.at[pl.ds(rbase + FB, FB)])
        pltpu.sync_copy(rows1.at[pl.ds(0, 64)], acc.at[pl.ds(rbase + 2 * FB, 64)])

        @pl.when(s == 0)
        def _():
            pltpu.sync_copy(rows1.at[pl.ds(0, 8)], gsp.at[pl.ds(TRASH, 8)])

        @pl.when(s == 8)
        def _():
            pltpu.sync_copy(rows1.at[pl.ds(0, 8)], gsp.at[pl.ds(TRASH + 8, 8)])

        def stage_g(r):
            for (off, sz) in ((0, FB), (FB, FB), (2 * FB, 64)):
                pltpu.sync_copy(
                    g_hbm.at[pl.ds(r * HALF + rbase + off, sz)],
                    rows0.at[pl.ds(0, sz)])
                pltpu.sync_copy(
                    rows0.at[pl.ds(0, sz)], gsp.at[pl.ds(rbase + off, sz)])

        ebase = s * (NSB * SB)

        def stage(sb_idx, par):
            pltpu.async_copy(
                src_hbm.at[pl.ds(ebase + sb_idx * SB, SB)], ssrcs[par], si)
            pltpu.async_copy(
                dst_hbm.at[pl.ds(ebase + sb_idx * SB, SB)], sdsts[par], si)

        def stage_wait(sb_idx, par):
            pltpu.make_async_copy(
                src_hbm.at[pl.ds(ebase + sb_idx * SB, SB)], ssrcs[par], si
            ).wait()
            pltpu.make_async_copy(
                dst_hbm.at[pl.ds(ebase + sb_idx * SB, SB)], sdsts[par], si
            ).wait()

        def run_superbatch(par, srcbase):
            ssrc = ssrcs[par]
            sdst = sdsts[par]
            tloc = TRASH + s
            for b in range(SB):
                for kk in range(B // 16):
                    cs = pl.ds(kk * 16, 16)
                    sv = ssrc[b, cs]
                    inr = (sv >= srcbase) & (sv < srcbase + HALF)
                    ssrc[b, cs] = jnp.where(inr, sv - srcbase, tloc)
                    dv = sdst[b, cs]
                    ind = (dv >= dstbase) & (dv < dstbase + HALF)
                    sdst[b, cs] = jnp.where(ind, dv - dstbase, tloc)

            def gissue(b):
                return pltpu.async_copy(
                    gsp.at[ssrc.at[b]], rb[b % 2], sg[b % 2])

            def sissue(b):
                return pltpu.async_copy(
                    rb[b % 2], acc.at[sdst.at[b]], ss[b % 2], add=True)

            gd = {0: gissue(0)}
            sd = {}
            for b in range(SB):
                if b + 1 < SB:
                    if b - 1 >= 0:
                        sd[b - 1].wait()
                    gd[b + 1] = gissue(b + 1)
                gd[b].wait()
                sd[b] = sissue(b)
            sd[SB - 2].wait()
            sd[SB - 1].wait()

        nhalf = NSB // 2

        def edge_loop(srcbase):
            stage(0, 0)

            def pair(m, _):
                stage_wait(2 * m, 0)
                stage(2 * m + 1, 1)
                run_superbatch(0, srcbase)
                stage_wait(2 * m + 1, 1)

                @pl.when(m < nhalf - 1)
                def _():
                    stage(2 * m + 2, 0)

                run_superbatch(1, srcbase)
                return 0

            lax.fori_loop(0, nhalf, pair, 0)

        stage_g(0)
        plsc.subcore_barrier()
        edge_loop(0)
        plsc.subcore_barrier()
        stage_g(1)
        plsc.subcore_barrier()
        edge_loop(HALF)
        plsc.subcore_barrier()

        def fin(off, sz):
            al = pl.ds(rbase + off, sz)
            gl = pl.ds(dstbase + rbase + off, sz)
            pltpu.sync_copy(acc.at[al], rows0.at[pl.ds(0, sz)])
            pltpu.sync_copy(g_hbm.at[gl], rows1.at[pl.ds(0, sz)])

            def fadd(i, _):
                for kk in range(8):
                    cs = pl.ds(kk * 16, 16)
                    rows0[i, cs] = rows0[i, cs] + rows1[i, cs]
                return 0

            lax.fori_loop(0, sz, fadd, 0)
            pltpu.sync_copy(scale_hbm.at[gl], rows1.at[pl.ds(0, sz)])

            def fmul(i, _):
                sc = rows1[i, :][0]
                for kk in range(8):
                    cs = pl.ds(kk * 16, 16)
                    rows0[i, cs] = sc * rows0[i, cs]
                return 0

            lax.fori_loop(0, sz, fmul, 0)
            pltpu.sync_copy(rows0.at[pl.ds(0, sz)], out_hbm.at[gl])

        fin(0, FB)
        fin(FB, FB)
        fin(2 * FB, 64)

    return k


def _dinv_tc_call(deg):
    BM = 1280

    def body(deg_ref, dinv_ref, d2_ref):
        y = lax.rsqrt(deg_ref[...])
        dinv_ref[...] = y
        d2_ref[...] = y * y

    return pl.pallas_call(
        body,
        out_shape=(
            jax.ShapeDtypeStruct((NPAD, 128), jnp.float32),
            jax.ShapeDtypeStruct((NPAD, 128), jnp.float32),
        ),
        grid=(NPAD // BM,),
        in_specs=[pl.BlockSpec((BM, 128), lambda i: (i, 0))],
        out_specs=(
            pl.BlockSpec((BM, 128), lambda i: (i, 0)),
            pl.BlockSpec((BM, 128), lambda i: (i, 0)),
        ),
    )(deg)


def _tc_in_call(xpad, W1, b1, dinv_col):
    D = xpad.shape[1]
    H = W1.shape[1]
    BM = 320

    def body(x_ref, w_ref, b_ref, dv_ref, o_ref):
        h = jnp.dot(x_ref[...], w_ref[...], preferred_element_type=jnp.float32)
        o_ref[...] = dv_ref[...] * (h + b_ref[...])

    return pl.pallas_call(
        body,
        out_shape=jax.ShapeDtypeStruct((NPAD, H), jnp.float32),
        grid=(NPAD // BM,),
        in_specs=[
            pl.BlockSpec((BM, D), lambda i: (i, 0)),
            pl.BlockSpec((D, H), lambda i: (0, 0)),
            pl.BlockSpec((1, H), lambda i: (0, 0)),
            pl.BlockSpec((BM, 1), lambda i: (i, 0)),
        ],
        out_specs=pl.BlockSpec((BM, H), lambda i: (i, 0)),
    )(xpad, W1, b1, dinv_col)


def _tc_out_call(g30, Wc, bc, dinv_col):
    H = g30.shape[1]
    C = Wc.shape[1]
    BM = 320

    def body(g_ref, w_ref, b_ref, dv_ref, o_ref):
        dv = dv_ref[...]
        h = jax.nn.relu(g_ref[...] / dv)
        q = dv * (jnp.dot(h, w_ref[...], preferred_element_type=jnp.float32)
                  + b_ref[...])
        o_ref[...] = jnp.concatenate(
            [q, jnp.zeros((BM, H - C), jnp.float32)], axis=1)

    return pl.pallas_call(
        body,
        out_shape=jax.ShapeDtypeStruct((NPAD, H), jnp.float32),
        grid=(NPAD // BM,),
        in_specs=[
            pl.BlockSpec((BM, H), lambda i: (i, 0)),
            pl.BlockSpec((H, C), lambda i: (0, 0)),
            pl.BlockSpec((1, C), lambda i: (0, 0)),
            pl.BlockSpec((BM, 1), lambda i: (i, 0)),
        ],
        out_specs=pl.BlockSpec((BM, H), lambda i: (i, 0)),
    )(g30, Wc, bc, dinv_col)


def kernel(x, edge_index, prop_nums, W1, b1, Wc, bc):
    C = Wc.shape[1]
    E = edge_index.shape[1]
    src = edge_index[0].astype(jnp.int32)
    dst = edge_index[1].astype(jnp.int32)
    pad = E_PAD - E
    srcp = jnp.concatenate([src, jnp.zeros((pad,), jnp.int32)])
    dstp = jnp.concatenate([dst, jnp.full((pad,), N, jnp.int32)])
    src2d = srcp.reshape(EROWS, B)
    dst2d = dstp.reshape(EROWS, B)
    xpad = jnp.pad(x, ((0, NPAD - N), (0, 0)))
    b1r = b1.reshape(1, -1)
    bcr = bc.reshape(1, -1)

    prop = _make_prop()

    def phase_deg(g, dv, d2):
        return (jnp.ones((NPAD, 128), jnp.float32),
                jnp.ones((NPAD, 128), jnp.float32), dv, d2)

    def phase_init(g, dv, d2):
        dv2, d22 = _dinv_tc_call(g)
        gin = _tc_in_call(xpad, W1, b1r, dv2[:, :1])
        return gin, d22, dv2, d22

    def phase_mid(g, dv, d2):
        return g, d2, dv, d2

    def phase_last(g, dv, d2):
        qpad = _tc_out_call(g, Wc, bcr, dv[:, :1])
        return qpad, dv, dv, d2

    def loop_body(i, carry):
        g, dv, d2 = carry
        sel = jnp.where(i == 0, 0,
                        jnp.where(i == 1, 1,
                                  jnp.where(i == prop_nums + 1, 3, 2)))
        gin, scale, dv, d2 = lax.switch(
            sel, [phase_deg, phase_init, phase_mid, phase_last], g, dv, d2)
        return prop(gin, src2d, dst2d, scale), dv, d2

    zscale = jnp.zeros((NPAD, 128), jnp.float32)
    out_full, _, _ = lax.fori_loop(
        0, prop_nums + 2, loop_body,
        (jnp.zeros((NPAD, 128), jnp.float32), zscale, zscale))
    return out_full[:N, :C]


# speedup vs baseline: 1.1226x; 1.1226x over previous
import functools

import jax
import jax.numpy as jnp
from jax import lax
from jax.experimental import pallas as pl
from jax.experimental.pallas import tpu as pltpu
from jax.experimental.pallas import tpu_sc as plsc

N = 10000
NPAD = 10240
HALF = NPAD // 2
SPR = HALF // 16
GROWS = HALF + 16
TRASH = HALF
FB = 128

B = 128
SB = 16
NSB = 10
EPT = B * SB * NSB
E_PAD = 16 * EPT
EROWS = E_PAD // B

_mesh = lambda: plsc.VectorSubcoreMesh(core_axis_name="c", subcore_axis_name="s")


def _make_prop():
    @functools.partial(
        pl.kernel,
        out_type=jax.ShapeDtypeStruct((NPAD, 128), jnp.float32),
        mesh=_mesh(),
        scratch_types=[
            pltpu.VMEM_SHARED((GROWS, 128), jnp.float32),
            pltpu.VMEM_SHARED((GROWS, 128), jnp.float32),
            pltpu.VMEM((SB, B), jnp.int32),
            pltpu.VMEM((SB, B), jnp.int32),
            pltpu.VMEM((SB, B), jnp.int32),
            pltpu.VMEM((SB, B), jnp.int32),
            pltpu.VMEM((B, 128), jnp.float32),
            pltpu.VMEM((B, 128), jnp.float32),
            pltpu.SemaphoreType.DMA,
            pltpu.SemaphoreType.DMA,
            pltpu.SemaphoreType.DMA,
            pltpu.SemaphoreType.DMA,
            pltpu.SemaphoreType.DMA,
        ],
    )
    def k(g_hbm, src_hbm, dst_hbm, scale_hbm, out_hbm,
          gsp, acc, ssrc0, ssrc1, sdst0, sdst1,
          rows0, rows1, sg0, sg1, ss0, ss1, si):
        c = lax.axis_index("c")
        s = lax.axis_index("s")
        rb = (rows0, rows1)
        sg = (sg0, sg1)
        ss = (ss0, ss1)
        ssrcs = (ssrc0, ssrc1)
        sdsts = (sdst0, sdst1)
        rbase = s * SPR
        dstbase = c * HALF

        def zrow(i, _):
            for kk in range(8):
                rows1[i, pl.ds(kk * 16, 16)] = jnp.zeros((16,), jnp.float32)
            return 0

        lax.fori_loop(0, FB, zrow, 0)
        pltpu.sync_copy(rows1, acc.at[pl.ds(rbase, FB)])
        pltpu.sync_copy(rows1, acc.at[pl.ds(rbase + FB, FB)])
        pltpu.sync_copy(rows1.at[pl.ds(0, 64)], acc.at[pl.ds(rbase + 2 * FB, 64)])

        @pl.when(s == 0)
        def _():
            pltpu.sync_copy(rows1.at[pl.ds(0, 8)], gsp.at[pl.ds(TRASH, 8)])

        @pl.when(s == 8)
        def _():
            pltpu.sync_copy(rows1.at[pl.ds(0, 8)], gsp.at[pl.ds(TRASH + 8, 8)])

        def stage_g(r):
            for (off, sz) in ((0, FB), (FB, FB), (2 * FB, 64)):
                pltpu.sync_copy(
                    g_hbm.at[pl.ds(r * HALF + rbase + off, sz)],
                    rows0.at[pl.ds(0, sz)])
                pltpu.sync_copy(
                    rows0.at[pl.ds(0, sz)], gsp.at[pl.ds(rbase + off, sz)])

        ebase = s * (NSB * SB)

        def stage(sb_idx, par):
            pltpu.async_copy(
                src_hbm.at[pl.ds(ebase + sb_idx * SB, SB)], ssrcs[par], si)
            pltpu.async_copy(
                dst_hbm.at[pl.ds(ebase + sb_idx * SB, SB)], sdsts[par], si)

        def stage_wait(sb_idx, par):
            pltpu.make_async_copy(
                src_hbm.at[pl.ds(ebase + sb_idx * SB, SB)], ssrcs[par], si
            ).wait()
            pltpu.make_async_copy(
                dst_hbm.at[pl.ds(ebase + sb_idx * SB, SB)], sdsts[par], si
            ).wait()

        def run_superbatch(par, srcbase):
            ssrc = ssrcs[par]
            sdst = sdsts[par]
            tloc = TRASH + s
            for b in range(SB):
                for kk in range(B // 16):
                    cs = pl.ds(kk * 16, 16)
                    sv = ssrc[b, cs]
                    inr = (sv >= srcbase) & (sv < srcbase + HALF)
                    ssrc[b, cs] = jnp.where(inr, sv - srcbase, tloc)
                    dv = sdst[b, cs]
                    ind = (dv >= dstbase) & (dv < dstbase + HALF)
                    sdst[b, cs] = jnp.where(ind, dv - dstbase, tloc)

            def gissue(b):
                return pltpu.async_copy(
                    gsp.at[ssrc.at[b]], rb[b % 2], sg[b % 2])

            def sissue(b):
                return pltpu.async_copy(
                    rb[b % 2], acc.at[sdst.at[b]], ss[b % 2], add=True)

            gd = {0: gissue(0)}
            sd = {}
            for b in range(SB):
                gd[b].wait()
                sd[b] = sissue(b)
                if b + 1 < SB:
                    if b - 1 >= 0:
                        sd[b - 1].wait()
                    gd[b + 1] = gissue(b + 1)
            sd[SB - 2].wait()
            sd[SB - 1].wait()

        nhalf = NSB // 2

        def edge_loop(srcbase):
            stage(0, 0)

            def pair(m, _):
                stage_wait(2 * m, 0)
                stage(2 * m + 1, 1)
                run_superbatch(0, srcbase)
                stage_wait(2 * m + 1, 1)

                @pl.when(m < nhalf - 1)
                def _():
                    stage(2 * m + 2, 0)

                run_superbatch(1, srcbase)
                return 0

            lax.fori_loop(0, nhalf, pair, 0)

        stage_g(0)
        plsc.subcore_barrier()
        edge_loop(0)
        plsc.subcore_barrier()
        stage_g(1)
        plsc.subcore_barrier()
        edge_loop(HALF)
        plsc.subcore_barrier()

        def fin(off, sz):
            al = pl.ds(rbase + off, sz)
            gl = pl.ds(dstbase + rbase + off, sz)
            pltpu.sync_copy(acc.at[al], rows0.at[pl.ds(0, sz)])
            pltpu.sync_copy(g_hbm.at[gl], rows1.at[pl.ds(0, sz)])

            def fadd(i, _):
                for kk in range(8):
                    cs = pl.ds(kk * 16, 16)
                    rows0[i, cs] = rows0[i, cs] + rows1[i, cs]
                return 0

            lax.fori_loop(0, sz, fadd, 0)
            pltpu.sync_copy(scale_hbm.at[gl], rows1.at[pl.ds(0, sz)])

            def fmul(i, _):
                sc = rows1[i, :][0]
                for kk in range(8):
                    cs = pl.ds(kk * 16, 16)
                    rows0[i, cs] = sc * rows0[i, cs]
                return 0

            lax.fori_loop(0, sz, fmul, 0)
            pltpu.sync_copy(rows0.at[pl.ds(0, sz)], out_hbm.at[gl])

        fin(0, FB)
        fin(FB, FB)
        fin(2 * FB, 64)

    return k


def _dinv_tc_call(deg):
    BM = 1280

    def body(deg_ref, dinv_ref, d2_ref):
        y = lax.rsqrt(deg_ref[...])
        dinv_ref[...] = y
        d2_ref[...] = y * y

    return pl.pallas_call(
        body,
        out_shape=(
            jax.ShapeDtypeStruct((NPAD, 128), jnp.float32),
            jax.ShapeDtypeStruct((NPAD, 128), jnp.float32),
        ),
        grid=(NPAD // BM,),
        in_specs=[pl.BlockSpec((BM, 128), lambda i: (i, 0))],
        out_specs=(
            pl.BlockSpec((BM, 128), lambda i: (i, 0)),
            pl.BlockSpec((BM, 128), lambda i: (i, 0)),
        ),
    )(deg)


def _tc_in_call(xpad, W1, b1, dinv_col):
    D = xpad.shape[1]
    H = W1.shape[1]
    BM = 320

    def body(x_ref, w_ref, b_ref, dv_ref, o_ref):
        h = jnp.dot(x_ref[...], w_ref[...], preferred_element_type=jnp.float32)
        o_ref[...] = dv_ref[...] * (h + b_ref[...])

    return pl.pallas_call(
        body,
        out_shape=jax.ShapeDtypeStruct((NPAD, H), jnp.float32),
        grid=(NPAD // BM,),
        in_specs=[
            pl.BlockSpec((BM, D), lambda i: (i, 0)),
            pl.BlockSpec((D, H), lambda i: (0, 0)),
            pl.BlockSpec((1, H), lambda i: (0, 0)),
            pl.BlockSpec((BM, 1), lambda i: (i, 0)),
        ],
        out_specs=pl.BlockSpec((BM, H), lambda i: (i, 0)),
    )(xpad, W1, b1, dinv_col)


def _tc_out_call(g30, Wc, bc, dinv_col):
    H = g30.shape[1]
    C = Wc.shape[1]
    BM = 320

    def body(g_ref, w_ref, b_ref, dv_ref, o_ref):
        dv = dv_ref[...]
        h = jax.nn.relu(g_ref[...] / dv)
        q = dv * (jnp.dot(h, w_ref[...], preferred_element_type=jnp.float32)
                  + b_ref[...])
        o_ref[...] = jnp.concatenate(
            [q, jnp.zeros((BM, H - C), jnp.float32)], axis=1)

    return pl.pallas_call(
        body,
        out_shape=jax.ShapeDtypeStruct((NPAD, H), jnp.float32),
        grid=(NPAD // BM,),
        in_specs=[
            pl.BlockSpec((BM, H), lambda i: (i, 0)),
            pl.BlockSpec((H, C), lambda i: (0, 0)),
            pl.BlockSpec((1, C), lambda i: (0, 0)),
            pl.BlockSpec((BM, 1), lambda i: (i, 0)),
        ],
        out_specs=pl.BlockSpec((BM, H), lambda i: (i, 0)),
    )(g30, Wc, bc, dinv_col)


def kernel(x, edge_index, prop_nums, W1, b1, Wc, bc):
    C = Wc.shape[1]
    E = edge_index.shape[1]
    src = edge_index[0].astype(jnp.int32)
    dst = edge_index[1].astype(jnp.int32)
    pad = E_PAD - E
    srcp = jnp.concatenate([src, jnp.zeros((pad,), jnp.int32)])
    dstp = jnp.concatenate([dst, jnp.full((pad,), N, jnp.int32)])
    src2d = srcp.reshape(EROWS, B)
    dst2d = dstp.reshape(EROWS, B)
    xpad = jnp.pad(x, ((0, NPAD - N), (0, 0)))
    b1r = b1.reshape(1, -1)
    bcr = bc.reshape(1, -1)

    prop = _make_prop()

    def phase_deg(g, dv, d2):
        return (jnp.ones((NPAD, 128), jnp.float32),
                jnp.ones((NPAD, 128), jnp.float32), dv, d2)

    def phase_init(g, dv, d2):
        dv2, d22 = _dinv_tc_call(g)
        gin = _tc_in_call(xpad, W1, b1r, dv2[:, :1])
        return gin, d22, dv2, d22

    def phase_mid(g, dv, d2):
        return g, d2, dv, d2

    def phase_last(g, dv, d2):
        qpad = _tc_out_call(g, Wc, bcr, dv[:, :1])
        return qpad, dv, dv, d2

    def loop_body(i, carry):
        g, dv, d2 = carry
        sel = jnp.where(i == 0, 0,
                        jnp.where(i == 1, 1,
                                  jnp.where(i == prop_nums + 1, 3, 2)))
        gin, scale, dv, d2 = lax.switch(
            sel, [phase_deg, phase_init, phase_mid, phase_last], g, dv, d2)
        return prop(gin, src2d, dst2d, scale), dv, d2

    zscale = jnp.zeros((NPAD, 128), jnp.float32)
    out_full, _, _ = lax.fori_loop(
        0, prop_nums + 2, loop_body,
        (jnp.zeros((NPAD, 128), jnp.float32), zscale, zscale))
    return out_full[:N, :C]
